# baseline (speedup 1.0000x reference)
import jax
import jax.numpy as jnp
from jax.experimental import pallas as pl
from jax.experimental.pallas import tpu as pltpu


def _swish(x):
    return 0.5 * x * (1.0 + jnp.tanh(0.5 * x))


_TM = 8


def _body(p_ref, v_ref, w1_ref, b1_ref, w2_ref, b2_ref, w3_ref,
          b3_ref, wp_ref, bp_ref, o_ref):
    p = p_ref[...]
    s = p[:, 6:7]
    h = _swish(p @ w1_ref[...] + b1_ref[...])
    h = _swish(h @ w2_ref[...] + b2_ref[...])
    w = _swish(h @ w3_ref[...] + b3_ref[...])
    w = w * s
    v = v_ref[0]
    cin = w.shape[-1]
    acc = jnp.sum(w.reshape(_TM, -1, cin) * v[None], axis=1)
    o_ref[...] = acc @ wp_ref[...] + bp_ref[...]


def kernel(pairs_ab, values, mask, W1, b1, W2, b2, W3, b3, Wp, bp):
    B, M, N, D = pairs_ab.shape
    Cin = values.shape[-1]
    Cout = Wp.shape[-1]
    H = W1.shape[-1]
    BM = B * M

    vals_masked = jnp.where(mask[:, :, None], values, 0.0)
    dists = jnp.linalg.norm(pairs_ab, axis=-1)
    s = (dists < 1.0).astype(jnp.float32).reshape(BM * N, 1)
    p_flat = pairs_ab.reshape(BM * N, D)
    p_aug = jnp.concatenate(
        [p_flat, s, jnp.zeros((BM * N, 1), jnp.float32)], axis=1)
    w1_pad = jnp.zeros((D + 2, H), jnp.float32).at[:D].set(W1)

    grid = (BM // _TM,)
    out = pl.pallas_call(
        _body,
        grid=grid,
        in_specs=[
            pl.BlockSpec((_TM * N, D + 2), lambda i: (i, 0)),
            pl.BlockSpec((1, N, Cin), lambda i: (i * _TM // M, 0, 0)),
            pl.BlockSpec((D + 2, H), lambda i: (0, 0)),
            pl.BlockSpec((1, H), lambda i: (0, 0)),
            pl.BlockSpec((H, H), lambda i: (0, 0)),
            pl.BlockSpec((1, H), lambda i: (0, 0)),
            pl.BlockSpec((H, Cin), lambda i: (0, 0)),
            pl.BlockSpec((1, Cin), lambda i: (0, 0)),
            pl.BlockSpec((Cin, Cout), lambda i: (0, 0)),
            pl.BlockSpec((1, Cout), lambda i: (0, 0)),
        ],
        out_specs=pl.BlockSpec((_TM, Cout), lambda i: (i, 0)),
        out_shape=jax.ShapeDtypeStruct((BM, Cout), jnp.float32),
        compiler_params=pltpu.CompilerParams(
            dimension_semantics=("arbitrary",),
        ),
    )(p_aug, vals_masked, w1_pad, b1.reshape(1, H), W2, b2.reshape(1, H),
      W3, b3.reshape(1, Cin), Wp, bp.reshape(1, Cout))

    out = jnp.where(mask.reshape(BM, 1), out, bp[None, :]).reshape(B, M, Cout)
    return (pairs_ab, out, mask)


# speedup vs baseline: 1.1635x; 1.1635x over previous
import jax
import jax.numpy as jnp
from jax.experimental import pallas as pl
from jax.experimental.pallas import tpu as pltpu


def _swish(x):
    return 0.5 * x * (1.0 + jnp.tanh(0.5 * x))


_TM = 16


def _body(p_ref, v_ref, w1_ref, b1_ref, w2_ref, b2_ref, w3_ref,
          wp_ref, bp_ref, o_ref):
    p = p_ref[...]
    d = jnp.sqrt(jnp.sum(p * p, axis=1, keepdims=True))
    s = jnp.where(d < 1.0, 1.0, 0.0)
    h = _swish(p @ w1_ref[...] + b1_ref[...])
    h = _swish(h @ w2_ref[...] + b2_ref[...])
    ones = jnp.ones((h.shape[0], 1), jnp.float32)
    h = jnp.concatenate([h, ones], axis=1)
    w = _swish(h @ w3_ref[...])
    w = w * s
    v = v_ref[0]
    cin = w.shape[-1]
    acc = jnp.sum(w.reshape(_TM, -1, cin) * v[None], axis=1)
    o_ref[...] = acc @ wp_ref[...] + bp_ref[...]


def kernel(pairs_ab, values, mask, W1, b1, W2, b2, W3, b3, Wp, bp):
    B, M, N, D = pairs_ab.shape
    Cin = values.shape[-1]
    Cout = Wp.shape[-1]
    H = W1.shape[-1]
    BM = B * M

    vals_masked = jnp.where(mask[:, :, None], values, 0.0)
    p_flat = pairs_ab.reshape(BM * N, D)
    w3_aug = jnp.concatenate([W3, b3.reshape(1, Cin)], axis=0)

    grid = (BM // _TM,)
    out = pl.pallas_call(
        _body,
        grid=grid,
        in_specs=[
            pl.BlockSpec((_TM * N, D), lambda i: (i, 0)),
            pl.BlockSpec((1, N, Cin), lambda i: (i * _TM // M, 0, 0)),
            pl.BlockSpec((D, H), lambda i: (0, 0)),
            pl.BlockSpec((1, H), lambda i: (0, 0)),
            pl.BlockSpec((H, H), lambda i: (0, 0)),
            pl.BlockSpec((1, H), lambda i: (0, 0)),
            pl.BlockSpec((H + 1, Cin), lambda i: (0, 0)),
            pl.BlockSpec((Cin, Cout), lambda i: (0, 0)),
            pl.BlockSpec((1, Cout), lambda i: (0, 0)),
        ],
        out_specs=pl.BlockSpec((_TM, Cout), lambda i: (i, 0)),
        out_shape=jax.ShapeDtypeStruct((BM, Cout), jnp.float32),
        compiler_params=pltpu.CompilerParams(
            dimension_semantics=("arbitrary",),
        ),
    )(p_flat, vals_masked, W1, b1.reshape(1, H), W2, b2.reshape(1, H),
      w3_aug, Wp, bp.reshape(1, Cout))

    out = jnp.where(mask.reshape(BM, 1), out, bp[None, :]).reshape(B, M, Cout)
    return (pairs_ab, out, mask)


# speedup vs baseline: 1.6549x; 1.4224x over previous
import jax
import jax.numpy as jnp
from jax.experimental import pallas as pl
from jax.experimental.pallas import tpu as pltpu


def _swish(x):
    return 0.5 * x * (1.0 + jnp.tanh(0.5 * x))


_TM = 16


def _body(p_ref, s_ref, v_ref, w1_ref, b1_ref, w2_ref, b2_ref, w3_ref,
          wp_ref, bp_ref, o_ref):
    p = p_ref[...]
    h = _swish(p @ w1_ref[...] + b1_ref[...])
    h = _swish(h @ w2_ref[...] + b2_ref[...])
    ones = jnp.ones((h.shape[0], 1), jnp.float32)
    h = jnp.concatenate([h, ones], axis=1)
    w = _swish(h @ w3_ref[...])
    v = v_ref[0]
    n = v.shape[0]
    cin = w.shape[-1]
    u = w.reshape(_TM, n, cin) * v[None]
    s = s_ref[...]
    acc = jnp.concatenate(
        [s[t:t + 1, :] @ u[t] for t in range(_TM)], axis=0)
    o_ref[...] = acc @ wp_ref[...] + bp_ref[...]


def kernel(pairs_ab, values, mask, W1, b1, W2, b2, W3, b3, Wp, bp):
    B, M, N, D = pairs_ab.shape
    Cin = values.shape[-1]
    Cout = Wp.shape[-1]
    H = W1.shape[-1]
    BM = B * M

    vals_masked = jnp.where(mask[:, :, None], values, 0.0)
    s = (jnp.linalg.norm(pairs_ab, axis=-1) < 1.0)
    s = s.astype(jnp.float32).reshape(BM, N)
    p_flat = pairs_ab.reshape(BM * N, D)
    w3_aug = jnp.concatenate([W3, b3.reshape(1, Cin)], axis=0)

    grid = (BM // _TM,)
    out = pl.pallas_call(
        _body,
        grid=grid,
        in_specs=[
            pl.BlockSpec((_TM * N, D), lambda i: (i, 0)),
            pl.BlockSpec((_TM, N), lambda i: (i, 0)),
            pl.BlockSpec((1, N, Cin), lambda i: (i * _TM // M, 0, 0)),
            pl.BlockSpec((D, H), lambda i: (0, 0)),
            pl.BlockSpec((1, H), lambda i: (0, 0)),
            pl.BlockSpec((H, H), lambda i: (0, 0)),
            pl.BlockSpec((1, H), lambda i: (0, 0)),
            pl.BlockSpec((H + 1, Cin), lambda i: (0, 0)),
            pl.BlockSpec((Cin, Cout), lambda i: (0, 0)),
            pl.BlockSpec((1, Cout), lambda i: (0, 0)),
        ],
        out_specs=pl.BlockSpec((_TM, Cout), lambda i: (i, 0)),
        out_shape=jax.ShapeDtypeStruct((BM, Cout), jnp.float32),
        compiler_params=pltpu.CompilerParams(
            dimension_semantics=("arbitrary",),
        ),
    )(p_flat, s, vals_masked, W1, b1.reshape(1, H), W2, b2.reshape(1, H),
      w3_aug, Wp, bp.reshape(1, Cout))

    out = jnp.where(mask.reshape(BM, 1), out, bp[None, :]).reshape(B, M, Cout)
    return (pairs_ab, out, mask)
